# baseline (speedup 1.0000x reference)
import jax
import jax.numpy as jnp
from jax.experimental import pallas as pl
from jax.experimental.pallas import tpu as pltpu

B = 16
T = 512
D = 256
H = 256
TA = 64
TC = 64
NCH = T // TC
G4 = 4 * H


def _lstm_gates(g, c):
    i = jax.nn.sigmoid(g[:, 0 * H:1 * H])
    f = jax.nn.sigmoid(g[:, 1 * H:2 * H])
    gg = jnp.tanh(g[:, 2 * H:3 * H])
    o = jax.nn.sigmoid(g[:, 3 * H:4 * H])
    c2 = f * c + i * gg
    h2 = o * jnp.tanh(c2)
    return h2, c2


def _vlslstm_kernel(x_ref, len_in_ref, len_ar_ref,
                    w0_ref, wh0_ref, wi1_ref, wh1_ref, b0_ref, b1_ref,
                    out_tf_ref, out_ar_ref,
                    pre_ref, gi1_ref, h0c_ref):
    f32 = jnp.float32
    lens_in = len_in_ref[...]
    lens_ar = len_ar_ref[...]
    wh0 = wh0_ref[...]
    wh1 = wh1_ref[...]
    b0 = b0_ref[...]
    b1 = b1_ref[...]

    zero = jnp.zeros((B, H), dtype=f32)

    def compute_pre(ch):
        xc = x_ref[pl.ds(ch * TC, TC)].reshape(TC * B, D)
        pre = jnp.dot(xc, w0_ref[...], preferred_element_type=f32) + b0
        pre_ref[...] = pre.reshape(TC, B, G4)

    def compute_gi1():
        hc = h0c_ref[...].reshape(TC * B, H)
        gi1 = jnp.dot(hc, wi1_ref[...], preferred_element_type=f32) + b1
        gi1_ref[...] = gi1.reshape(TC, B, G4)

    def l0_step(base, t, h, c):
        g = pre_ref[pl.ds(t, 1)].reshape(B, G4) + jnp.dot(
            h.astype(jnp.bfloat16), wh0, preferred_element_type=f32)
        h2, c2 = _lstm_gates(g, c)
        h0c_ref[pl.ds(t, 1)] = h2.reshape(1, B, H)
        m = (base + t) < lens_in
        return jnp.where(m, h2, h), jnp.where(m, c2, c)

    def l1_step(base, t, h, c, seed):
        g = gi1_ref[pl.ds(t, 1)].reshape(B, G4) + jnp.dot(
            h.astype(jnp.bfloat16), wh1, preferred_element_type=f32)
        h2, c2 = _lstm_gates(g, c)
        tt = base + t
        m = tt < lens_in
        out = jnp.where(m, h2, 0.0)
        out_tf_ref[pl.ds(tt, 1)] = out.reshape(1, B, H)
        seed = jnp.where(lens_in == tt + 1, out, seed)
        return jnp.where(m, h2, h), jnp.where(m, c2, c), seed

    compute_pre(0)

    def pro_body(t, carry):
        h, c = carry
        return l0_step(0, t, h, c)

    h0, c0 = jax.lax.fori_loop(0, TC, pro_body, (zero, zero))
    compute_gi1()

    h1, c1, init_in = zero, zero, zero
    for ch in range(1, NCH):
        compute_pre(ch)
        base0 = ch * TC
        base1 = (ch - 1) * TC

        def duo_body(t, carry, base0=base0, base1=base1):
            h0, c0, h1, c1, seed = carry
            h0n, c0n = l0_step(base0, t, h0, c0)
            h1n, c1n, seed = l1_step(base1, t, h1, c1, seed)
            return h0n, c0n, h1n, c1n, seed

        h0, c0, h1, c1, init_in = jax.lax.fori_loop(
            0, TC, duo_body, (h0, c0, h1, c1, init_in))
        compute_gi1()

    def epi_body(t, carry, base=(NCH - 1) * TC):
        h, c, seed = carry
        return l1_step(base, t, h, c, seed)

    h1, c1, init_in = jax.lax.fori_loop(0, TC, epi_body, (h1, c1, init_in))

    w0 = w0_ref[...].astype(jnp.bfloat16)
    wi1 = wi1_ref[...].astype(jnp.bfloat16)

    def ar_step(t, carry):
        h0, c0, h1, c1, inp = carry
        g0 = (jnp.dot(inp.astype(jnp.bfloat16), w0, preferred_element_type=f32)
              + jnp.dot(h0.astype(jnp.bfloat16), wh0, preferred_element_type=f32) + b0)
        h20, c20 = _lstm_gates(g0, c0)
        g1 = (jnp.dot(h20.astype(jnp.bfloat16), wi1, preferred_element_type=f32)
              + jnp.dot(h1.astype(jnp.bfloat16), wh1, preferred_element_type=f32) + b1)
        h21, c21 = _lstm_gates(g1, c1)
        m = t < lens_ar
        out = jnp.where(m, h21, 0.0)
        out_ar_ref[pl.ds(t, 1)] = out.reshape(1, B, H)
        return (jnp.where(m, h20, h0), jnp.where(m, c20, c0),
                jnp.where(m, h21, h1), jnp.where(m, c21, c1), out)

    jax.lax.fori_loop(0, TA, ar_step, (h0, c0, h1, c1, init_in))


@jax.jit
def _run(xT, len_in, len_ar, w0, wh0, wi1, wh1, b0, b1):
    f32 = jnp.float32
    out_tf, out_ar = pl.pallas_call(
        _vlslstm_kernel,
        out_shape=[
            jax.ShapeDtypeStruct((T, B, H), f32),
            jax.ShapeDtypeStruct((TA, B, H), f32),
        ],
        scratch_shapes=[
            pltpu.VMEM((TC, B, G4), f32),
            pltpu.VMEM((TC, B, G4), f32),
            pltpu.VMEM((TC, B, H), f32),
        ],
    )(xT, len_in, len_ar, w0, wh0, wi1, wh1, b0, b1)
    return out_tf, out_ar


def kernel(x, lengths_in, lengths_aureg, mask_aureg,
           W_ih0, W_hh0, b_ih0, b_hh0, W_ih1, W_hh1, b_ih1, b_hh1):
    xT = jnp.transpose(x, (1, 0, 2))
    len_in = lengths_in.reshape(B, 1)
    len_ar = lengths_aureg.reshape(B, 1)
    w0 = W_ih0.T
    wh0 = W_hh0.T.astype(jnp.bfloat16)
    wi1 = W_ih1.T
    wh1 = W_hh1.T.astype(jnp.bfloat16)
    b0 = (b_ih0 + b_hh0).reshape(1, G4)
    b1 = (b_ih1 + b_hh1).reshape(1, G4)
    out_tf, out_ar = _run(xT, len_in, len_ar, w0, wh0, wi1, wh1, b0, b1)
    teafo = jnp.transpose(out_tf, (1, 0, 2))
    aureg = jnp.transpose(out_ar, (1, 0, 2))
    return (teafo, aureg)


# speedup vs baseline: 1.1114x; 1.1114x over previous
import jax
import jax.numpy as jnp
from jax.experimental import pallas as pl
from jax.experimental.pallas import tpu as pltpu

B = 16
T = 512
D = 256
H = 256
TA = 64
TC = 64
NCH = T // TC
G4 = 4 * H


def _lstm_gates(g, c):
    i = jax.nn.sigmoid(g[:, 0 * H:1 * H])
    f = jax.nn.sigmoid(g[:, 1 * H:2 * H])
    gg = jnp.tanh(g[:, 2 * H:3 * H])
    o = jax.nn.sigmoid(g[:, 3 * H:4 * H])
    c2 = f * c + i * gg
    h2 = o * jnp.tanh(c2)
    return h2, c2


def _vlslstm_kernel(x_ref, len_in_ref, len_ar_ref,
                    w0_ref, wh0_ref, wi1_ref, wh1_ref, b0_ref, b1_ref,
                    out_tf_ref, out_ar_ref,
                    pre_ref, gi1_ref, h0c_ref):
    f32 = jnp.float32
    lens_in = len_in_ref[...]
    lens_ar = len_ar_ref[...]
    wh0 = wh0_ref[...]
    wh1 = wh1_ref[...]
    b0 = b0_ref[...]
    b1 = b1_ref[...]

    zero = jnp.zeros((B, H), dtype=f32)

    def compute_pre(ch):
        xc = x_ref[pl.ds(ch * TC, TC)].reshape(TC * B, D)
        pre = jnp.dot(xc, w0_ref[...], preferred_element_type=f32) + b0
        pre_ref[...] = pre.reshape(TC, B, G4)

    def compute_gi1():
        hc = h0c_ref[...].reshape(TC * B, H)
        gi1 = jnp.dot(hc, wi1_ref[...], preferred_element_type=f32) + b1
        gi1_ref[...] = gi1.reshape(TC, B, G4)

    def l0_step(base, t, h, c):
        g = pre_ref[pl.ds(t, 1)].reshape(B, G4) + jnp.dot(
            h, wh0, preferred_element_type=f32)
        h2, c2 = _lstm_gates(g, c)
        h0c_ref[pl.ds(t, 1)] = h2.reshape(1, B, H)
        m = (base + t) < lens_in
        return jnp.where(m, h2, h), jnp.where(m, c2, c)

    def l1_step(base, t, h, c, seed):
        g = gi1_ref[pl.ds(t, 1)].reshape(B, G4) + jnp.dot(
            h, wh1, preferred_element_type=f32)
        h2, c2 = _lstm_gates(g, c)
        tt = base + t
        m = tt < lens_in
        out = jnp.where(m, h2, 0.0)
        out_tf_ref[pl.ds(tt, 1)] = out.reshape(1, B, H)
        seed = jnp.where(lens_in == tt + 1, out, seed)
        return jnp.where(m, h2, h), jnp.where(m, c2, c), seed

    compute_pre(0)

    def pro_body(t, carry):
        h, c = carry
        return l0_step(0, t, h, c)

    h0, c0 = jax.lax.fori_loop(0, TC, pro_body, (zero, zero))
    compute_gi1()

    h1, c1, init_in = zero, zero, zero
    for ch in range(1, NCH):
        compute_pre(ch)
        base0 = ch * TC
        base1 = (ch - 1) * TC

        def duo_body(k, carry, base0=base0, base1=base1):
            h0, c0, h1, c1, seed = carry
            t = k * 2
            h0, c0 = l0_step(base0, t, h0, c0)
            h1, c1, seed = l1_step(base1, t, h1, c1, seed)
            h0, c0 = l0_step(base0, t + 1, h0, c0)
            h1, c1, seed = l1_step(base1, t + 1, h1, c1, seed)
            return h0, c0, h1, c1, seed

        h0, c0, h1, c1, init_in = jax.lax.fori_loop(
            0, TC // 2, duo_body, (h0, c0, h1, c1, init_in))
        compute_gi1()

    def epi_body(t, carry, base=(NCH - 1) * TC):
        h, c, seed = carry
        return l1_step(base, t, h, c, seed)

    h1, c1, init_in = jax.lax.fori_loop(0, TC, epi_body, (h1, c1, init_in))

    w0 = w0_ref[...]
    wi1 = wi1_ref[...]

    def ar_step(t, carry):
        h0, c0, h1, c1, inp = carry
        g0 = (jnp.dot(inp, w0, preferred_element_type=f32)
              + jnp.dot(h0, wh0, preferred_element_type=f32) + b0)
        h20, c20 = _lstm_gates(g0, c0)
        g1 = (jnp.dot(h20, wi1, preferred_element_type=f32)
              + jnp.dot(h1, wh1, preferred_element_type=f32) + b1)
        h21, c21 = _lstm_gates(g1, c1)
        m = t < lens_ar
        out = jnp.where(m, h21, 0.0)
        out_ar_ref[pl.ds(t, 1)] = out.reshape(1, B, H)
        return (jnp.where(m, h20, h0), jnp.where(m, c20, c0),
                jnp.where(m, h21, h1), jnp.where(m, c21, c1), out)

    jax.lax.fori_loop(0, TA, ar_step, (h0, c0, h1, c1, init_in))


@jax.jit
def _run(xT, len_in, len_ar, w0, wh0, wi1, wh1, b0, b1):
    f32 = jnp.float32
    out_tf, out_ar = pl.pallas_call(
        _vlslstm_kernel,
        out_shape=[
            jax.ShapeDtypeStruct((T, B, H), f32),
            jax.ShapeDtypeStruct((TA, B, H), f32),
        ],
        scratch_shapes=[
            pltpu.VMEM((TC, B, G4), f32),
            pltpu.VMEM((TC, B, G4), f32),
            pltpu.VMEM((TC, B, H), f32),
        ],
    )(xT, len_in, len_ar, w0, wh0, wi1, wh1, b0, b1)
    return out_tf, out_ar


def kernel(x, lengths_in, lengths_aureg, mask_aureg,
           W_ih0, W_hh0, b_ih0, b_hh0, W_ih1, W_hh1, b_ih1, b_hh1):
    xT = jnp.transpose(x, (1, 0, 2))
    len_in = lengths_in.reshape(B, 1)
    len_ar = lengths_aureg.reshape(B, 1)
    w0 = W_ih0.T
    wh0 = W_hh0.T
    wi1 = W_ih1.T
    wh1 = W_hh1.T
    b0 = (b_ih0 + b_hh0).reshape(1, G4)
    b1 = (b_ih1 + b_hh1).reshape(1, G4)
    out_tf, out_ar = _run(xT, len_in, len_ar, w0, wh0, wi1, wh1, b0, b1)
    teafo = jnp.transpose(out_tf, (1, 0, 2))
    aureg = jnp.transpose(out_ar, (1, 0, 2))
    return (teafo, aureg)
